# baseline (speedup 1.0000x reference)
import functools

import jax
import jax.numpy as jnp
from jax import lax
from jax.experimental import pallas as pl
from jax.experimental.pallas import tpu as pltpu
from jax.experimental.pallas import tpu_sc as plsc

_D = _H = _W = 128
_C = 4
_N = _D * _H * _W
_NC, _NS, _L = 2, 16, 16
_NW = _NC * _NS
_VPW = _N // _NW
_CH = 512
_NCHUNK = _VPW // _CH
_NVR = _CH // _L
_NIDX = 2 * _CH
_DEPTH = 4
_BCH = 1024
_BNCHUNK = _VPW // _BCH
_BNVR = _BCH // _L
_UNROLL = 4
_UNROLL2 = 2
_BCHP = _BCH + 144


def _f(x):
    return jnp.float32(x)


def _sc_build_table(src2d):
    mesh = plsc.VectorSubcoreMesh(core_axis_name="c", subcore_axis_name="s",
                                  num_cores=_NC, num_subcores=_NS)

    @functools.partial(
        pl.kernel,
        out_type=jax.ShapeDtypeStruct((_N * 16,), jnp.float32),
        mesh=mesh,
        compiler_params=pltpu.CompilerParams(needs_layout_passes=False,
                                             use_tc_tiling_on_sc=False),
        scratch_types=[
            pltpu.VMEM((2, _C, _BCHP), jnp.float32),
            pltpu.VMEM((2, _BCH * 16), jnp.float32),
            pltpu.SemaphoreType.DMA((2,)),
            pltpu.SemaphoreType.DMA((2,)),
        ],
    )
    def k(src_hbm, tab_hbm, sbuf, tbuf, sem_i, sem_o):
        wid = lax.axis_index("s") * _NC + lax.axis_index("c")
        vbase = wid * _VPW
        lanes = lax.iota(jnp.int32, _L)
        lanes16 = lanes * 16

        def fire_in(t, slot):
            base = vbase + t * _BCH
            sbase = jnp.minimum(base, _N - _BCHP)
            pltpu.async_copy(src_hbm.at[:, pl.ds(sbase, _BCHP)],
                             sbuf.at[slot], sem_i.at[slot])

        def wait_in(slot):
            pltpu.make_async_copy(src_hbm.at[:, pl.ds(0, _BCHP)],
                                  sbuf.at[slot], sem_i.at[slot]).wait()

        def fire_out(t, slot):
            base = vbase + t * _BCH
            pltpu.async_copy(tbuf.at[slot],
                             tab_hbm.at[pl.ds(base * 16, _BCH * 16)],
                             sem_o.at[slot])

        def wait_out(slot):
            pltpu.make_async_copy(tbuf.at[slot],
                                  tab_hbm.at[pl.ds(0, _BCH * 16)],
                                  sem_o.at[slot]).wait()

        def scatter(t, slot):
            base = vbase + t * _BCH
            is_tail = base > _N - _BCHP
            shift = base - jnp.minimum(base, _N - _BCHP)

            @pl.when(jnp.logical_not(is_tail))
            def _():
                @plsc.parallel_loop(0, _BNVR, 1, unroll=_UNROLL)
                def _(j):
                    off = j * _L
                    ib = off * 16 + lanes16
                    for c in range(_C):
                        for q, dlt in enumerate((0, 1, 128, 129)):
                            v = sbuf[slot, c, pl.ds(off + dlt, _L)]
                            plsc.store_scatter(tbuf.at[slot],
                                               [ib + (q * 4 + c)], v)

            @pl.when(is_tail)
            def _():
                @plsc.parallel_loop(0, _BNVR, 1, unroll=1)
                def _(j):
                    off = j * _L
                    ib = off * 16 + lanes16
                    for c in range(_C):
                        for q, dlt in enumerate((0, 1, 128, 129)):
                            hidx = jnp.minimum(shift + off + dlt + lanes,
                                               _BCHP - 1)
                            v = plsc.load_gather(sbuf.at[slot, c], [hidx])
                            plsc.store_scatter(tbuf.at[slot],
                                               [ib + (q * 4 + c)], v)

        fire_in(0, 0)
        fire_in(1, 1)
        wait_in(0)
        scatter(0, 0)
        fire_out(0, 0)

        def body(t, _):
            s = t & 1
            ns = 1 - s
            wait_in(s)

            @pl.when(t >= 2)
            def _():
                wait_out(s)
            scatter(t, s)
            fire_out(t, s)

            @pl.when(t + 1 < _BNCHUNK)
            def _():
                fire_in(t + 1, ns)
            return ()

        lax.fori_loop(1, _BNCHUNK, body, (), unroll=False)
        wait_out((_BNCHUNK - 2) & 1)
        wait_out((_BNCHUNK - 1) & 1)

    return k(src2d)


def _sc_warp(table, flow3):
    mesh = plsc.VectorSubcoreMesh(core_axis_name="c", subcore_axis_name="s",
                                  num_cores=_NC, num_subcores=_NS)

    @functools.partial(
        pl.kernel,
        out_type=jax.ShapeDtypeStruct((_C, _N), jnp.float32),
        mesh=mesh,
        compiler_params=pltpu.CompilerParams(needs_layout_passes=False,
                                             use_tc_tiling_on_sc=False),
        scratch_types=[
            pltpu.VMEM((_DEPTH, 3, _CH), jnp.float32),
            pltpu.VMEM((_DEPTH, _NIDX), jnp.int32),
            pltpu.VMEM((_DEPTH, _NIDX, 16), jnp.float32),
            pltpu.VMEM((_DEPTH, 8, _CH), jnp.float32),
            pltpu.VMEM((2, _C, _CH), jnp.float32),
            pltpu.SemaphoreType.DMA((_DEPTH,)),
            pltpu.SemaphoreType.DMA((_DEPTH,)),
        ],
    )
    def k(table_hbm, flow_hbm, out_hbm, flow_v, idx_v, rows_v, w_v, out_v,
          sem_f, sem_g):
        wid = lax.axis_index("s") * _NC + lax.axis_index("c")
        vbase = wid * _VPW
        lanes = lax.iota(jnp.int32, _L)
        lanes_f = lanes.astype(jnp.float32)
        lane_s = lanes_f * _f(1.0 / 64.0)

        def coord(sbase, fl):
            x = sbase + fl
            i = ((x + _f(1.0)) * _f(128.0) - _f(1.0)) * _f(0.5)
            ic = jnp.minimum(jnp.maximum(i, _f(-4.0)), _f(132.0))
            t = ic.astype(jnp.int32).astype(jnp.float32)
            i0 = t - jnp.where(t > ic, _f(1.0), _f(0.0))
            w1 = ic - i0
            w0 = _f(1.0) - w1
            return i0, w0, w1

        def lohi(i0, w0, w1):
            m0 = (i0 >= _f(0.0)) & (i0 <= _f(127.0))
            wlo = jnp.where(m0, w0, jnp.where(i0 == _f(-1.0), w1, _f(0.0)))
            whi = jnp.where((i0 >= _f(0.0)) & (i0 <= _f(126.0)), w1, _f(0.0))
            r = jnp.minimum(jnp.maximum(i0, _f(0.0)),
                            _f(127.0)).astype(jnp.int32)
            return wlo, whi, r

        def stage1(slot, base):
            zb = base >> 14
            sz = (_f(2.0) * zb.astype(jnp.float32) + _f(1.0)) \
                * _f(1.0 / 128.0) - _f(1.0)
            sz_v = jnp.full((_L,), sz, jnp.float32)

            @plsc.parallel_loop(0, _NVR, 1, unroll=_UNROLL)
            def _(j):
                off = j * _L
                v0 = base + off
                xb = v0 & 127
                yb = (v0 >> 7) & 127
                sx = (_f(2.0) * xb.astype(jnp.float32) + _f(1.0)) \
                    * _f(1.0 / 128.0) - _f(1.0)
                sy = (_f(2.0) * yb.astype(jnp.float32) + _f(1.0)) \
                    * _f(1.0 / 128.0) - _f(1.0)
                sx_v = jnp.full((_L,), sx, jnp.float32) + lane_s
                sy_v = jnp.full((_L,), sy, jnp.float32)

                ix0, wx0, wx1 = coord(sx_v, flow_v[slot, 0, pl.ds(off, _L)])
                iy0, wy0, wy1 = coord(sy_v, flow_v[slot, 1, pl.ds(off, _L)])
                iz0, wz0, wz1 = coord(sz_v, flow_v[slot, 2, pl.ds(off, _L)])

                wxlo, wxhi, xr = lohi(ix0, wx0, wx1)
                wylo, wyhi, yr = lohi(iy0, wy0, wy1)

                wz0m = jnp.where((iz0 >= _f(0.0)) & (iz0 <= _f(127.0)),
                                 wz0, _f(0.0))
                wz1m = jnp.where((iz0 >= _f(-1.0)) & (iz0 <= _f(126.0)),
                                 wz1, _f(0.0))
                zr0 = jnp.minimum(jnp.maximum(iz0, _f(0.0)),
                                  _f(127.0)).astype(jnp.int32)
                zr1 = jnp.minimum(jnp.maximum(iz0 + _f(1.0), _f(0.0)),
                                  _f(127.0)).astype(jnp.int32)

                rbase = (yr << 7) + xr
                idx_v[slot, pl.ds(off, _L)] = (zr0 << 14) + rbase
                idx_v[slot, pl.ds(_CH + off, _L)] = (zr1 << 14) + rbase

                pq = (wylo * wxlo, wylo * wxhi, wyhi * wxlo, wyhi * wxhi)
                for q in range(4):
                    w_v[slot, q, pl.ds(off, _L)] = wz0m * pq[q]
                    w_v[slot, 4 + q, pl.ds(off, _L)] = wz1m * pq[q]

        def stage2(slot):
            @plsc.parallel_loop(0, _NVR, 1, unroll=_UNROLL2)
            def _(j):
                off = j * _L
                acc = [jnp.zeros((_L,), jnp.float32) for _ in range(_C)]
                for s in range(2):
                    ridx = s * _CH + off + lanes
                    for q in range(4):
                        w = w_v[slot, s * 4 + q, pl.ds(off, _L)]
                        for c in range(_C):
                            val = plsc.load_gather(
                                rows_v.at[slot],
                                [ridx, jnp.full((_L,), q * 4 + c, jnp.int32)])
                            acc[c] = acc[c] + w * val
                for c in range(_C):
                    out_v[slot & 1, c, pl.ds(off, _L)] = acc[c]

        def fire_flow(t, slot):
            base = vbase + t * _CH
            pltpu.async_copy(flow_hbm.at[:, pl.ds(base, _CH)],
                             flow_v.at[slot], sem_f.at[slot])

        def fire_gather(slot):
            pltpu.async_copy(table_hbm.at[idx_v.at[slot]], rows_v.at[slot],
                             sem_g.at[slot])

        def wait_flow(slot):
            pltpu.make_async_copy(flow_hbm.at[:, pl.ds(0, _CH)],
                                  flow_v.at[slot], sem_f.at[slot]).wait()

        def wait_gather(slot):
            pltpu.make_async_copy(table_hbm.at[idx_v.at[slot]],
                                  rows_v.at[slot], sem_g.at[slot]).wait()

        def finish(t, slot):
            wait_gather(slot)
            stage2(slot)
            base = vbase + t * _CH
            pltpu.sync_copy(out_v.at[slot & 1],
                            out_hbm.at[:, pl.ds(base, _CH)])

        for i in range(_DEPTH):
            fire_flow(i, i)
        for i in range(_DEPTH - 1):
            wait_flow(i)
            stage1(i, vbase + i * _CH)
            fire_gather(i)

        def chunk_body(t, _):
            s = t % _DEPTH
            ps = (t + 1) % _DEPTH
            wait_flow(s)
            stage1(s, vbase + t * _CH)
            fire_gather(s)
            finish(t - (_DEPTH - 1), ps)
            @pl.when(t + 1 < _NCHUNK)
            def _():
                fire_flow(t + 1, (t + 1) % _DEPTH)
            return ()

        lax.fori_loop(_DEPTH - 1, _NCHUNK, chunk_body, (), unroll=False)
        for r in range(_NCHUNK - _DEPTH + 1, _NCHUNK):
            finish(r, r % _DEPTH)

    return k(table, flow3)


def kernel(src, flow):
    src2d = src.reshape(_C, _N)
    flow3 = flow.reshape(3, _N)
    table = _sc_build_table(src2d).reshape(_N, 16)
    out = _sc_warp(table, flow3)
    return out.reshape(1, _C, _D, _H, _W)


# speedup vs baseline: 1.5726x; 1.5726x over previous
import functools

import jax
import jax.numpy as jnp
from jax import lax
from jax.experimental import pallas as pl
from jax.experimental.pallas import tpu as pltpu
from jax.experimental.pallas import tpu_sc as plsc

_D = _H = _W = 128
_C = 4
_N = _D * _H * _W
_NC, _NS, _L = 2, 16, 16
_NW = _NC * _NS
_VPW = _N // _NW
_CH = 512
_NCHUNK = _VPW // _CH
_NVR = _CH // _L
_NIDX = 2 * _CH
_DEPTH = 4
_BCH = 1024
_BNCHUNK = _VPW // _BCH
_BNVR = _BCH // _L
_UNROLL = 4
_UNROLL2 = 2
_BCHP = _BCH + 144


def _f(x):
    return jnp.float32(x)


def _sc_build_table(src2d):
    mesh = plsc.VectorSubcoreMesh(core_axis_name="c", subcore_axis_name="s",
                                  num_cores=_NC, num_subcores=_NS)

    @functools.partial(
        pl.kernel,
        out_type=jax.ShapeDtypeStruct((_N * 8,), jnp.float32),
        mesh=mesh,
        compiler_params=pltpu.CompilerParams(needs_layout_passes=False,
                                             use_tc_tiling_on_sc=False),
        scratch_types=[
            pltpu.VMEM((2, _C, _BCHP), jnp.float32),
            pltpu.VMEM((2, _BCH * 8), jnp.float32),
            pltpu.SemaphoreType.DMA((2,)),
            pltpu.SemaphoreType.DMA((2,)),
        ],
    )
    def k(src_hbm, tab_hbm, sbuf, tbuf, sem_i, sem_o):
        wid = lax.axis_index("s") * _NC + lax.axis_index("c")
        vbase = wid * _VPW
        lanes = lax.iota(jnp.int32, _L)
        lanes8 = lanes * 8

        def fire_in(t, slot):
            base = vbase + t * _BCH
            sbase = jnp.minimum(base, _N - _BCHP)
            pltpu.async_copy(src_hbm.at[:, pl.ds(sbase, _BCHP)],
                             sbuf.at[slot], sem_i.at[slot])

        def wait_in(slot):
            pltpu.make_async_copy(src_hbm.at[:, pl.ds(0, _BCHP)],
                                  sbuf.at[slot], sem_i.at[slot]).wait()

        def fire_out(t, slot):
            base = vbase + t * _BCH
            pltpu.async_copy(tbuf.at[slot],
                             tab_hbm.at[pl.ds(base * 8, _BCH * 8)],
                             sem_o.at[slot])

        def wait_out(slot):
            pltpu.make_async_copy(tbuf.at[slot],
                                  tab_hbm.at[pl.ds(0, _BCH * 8)],
                                  sem_o.at[slot]).wait()

        def scatter(t, slot):
            base = vbase + t * _BCH
            is_tail = base > _N - _BCHP
            shift = base - jnp.minimum(base, _N - _BCHP)

            @pl.when(jnp.logical_not(is_tail))
            def _():
                @plsc.parallel_loop(0, _BNVR, 1, unroll=_UNROLL)
                def _(j):
                    off = j * _L
                    ib = off * 8 + lanes8
                    for q, dlt in enumerate((0, 1, 128, 129)):
                        vs = [sbuf[slot, c, pl.ds(off + dlt, _L)]
                              for c in range(_C)]
                        for h in range(2):
                            pk = plsc.bitcast(
                                plsc.pack(vs[2 * h], vs[2 * h + 1],
                                          format=plsc.PackFormat.INTERLEAVED),
                                jnp.float32)
                            plsc.store_scatter(tbuf.at[slot],
                                               [ib + (q * 2 + h)], pk)

            @pl.when(is_tail)
            def _():
                @plsc.parallel_loop(0, _BNVR, 1, unroll=1)
                def _(j):
                    off = j * _L
                    ib = off * 8 + lanes8
                    for q, dlt in enumerate((0, 1, 128, 129)):
                        hidx = jnp.minimum(shift + off + dlt + lanes,
                                           _BCHP - 1)
                        vs = [plsc.load_gather(sbuf.at[slot, c], [hidx])
                              for c in range(_C)]
                        for h in range(2):
                            pk = plsc.bitcast(
                                plsc.pack(vs[2 * h], vs[2 * h + 1],
                                          format=plsc.PackFormat.INTERLEAVED),
                                jnp.float32)
                            plsc.store_scatter(tbuf.at[slot],
                                               [ib + (q * 2 + h)], pk)

        fire_in(0, 0)
        fire_in(1, 1)
        wait_in(0)
        scatter(0, 0)
        fire_out(0, 0)

        def body(t, _):
            s = t & 1
            ns = 1 - s
            wait_in(s)

            @pl.when(t >= 2)
            def _():
                wait_out(s)
            scatter(t, s)
            fire_out(t, s)

            @pl.when(t + 1 < _BNCHUNK)
            def _():
                fire_in(t + 1, ns)
            return ()

        lax.fori_loop(1, _BNCHUNK, body, (), unroll=False)
        wait_out((_BNCHUNK - 2) & 1)
        wait_out((_BNCHUNK - 1) & 1)

    return k(src2d)


def _sc_warp(table, flow3):
    mesh = plsc.VectorSubcoreMesh(core_axis_name="c", subcore_axis_name="s",
                                  num_cores=_NC, num_subcores=_NS)

    @functools.partial(
        pl.kernel,
        out_type=jax.ShapeDtypeStruct((_C, _N), jnp.float32),
        mesh=mesh,
        compiler_params=pltpu.CompilerParams(needs_layout_passes=False,
                                             use_tc_tiling_on_sc=False),
        scratch_types=[
            pltpu.VMEM((_DEPTH, 3, _CH), jnp.float32),
            pltpu.VMEM((_DEPTH, _NIDX), jnp.int32),
            pltpu.VMEM((_DEPTH, _NIDX, 8), jnp.float32),
            pltpu.VMEM((_DEPTH, 8, _CH), jnp.float32),
            pltpu.VMEM((2, _C, _CH), jnp.float32),
            pltpu.SemaphoreType.DMA((_DEPTH,)),
            pltpu.SemaphoreType.DMA((_DEPTH,)),
        ],
    )
    def k(table_hbm, flow_hbm, out_hbm, flow_v, idx_v, rows_v, w_v, out_v,
          sem_f, sem_g):
        wid = lax.axis_index("s") * _NC + lax.axis_index("c")
        vbase = wid * _VPW
        lanes = lax.iota(jnp.int32, _L)
        lanes_f = lanes.astype(jnp.float32)
        lane_s = lanes_f * _f(1.0 / 64.0)

        def coord(sbase, fl):
            x = sbase + fl
            i = ((x + _f(1.0)) * _f(128.0) - _f(1.0)) * _f(0.5)
            ic = jnp.minimum(jnp.maximum(i, _f(-4.0)), _f(132.0))
            t = ic.astype(jnp.int32).astype(jnp.float32)
            i0 = t - jnp.where(t > ic, _f(1.0), _f(0.0))
            w1 = ic - i0
            w0 = _f(1.0) - w1
            return i0, w0, w1

        def lohi(i0, w0, w1):
            m0 = (i0 >= _f(0.0)) & (i0 <= _f(127.0))
            wlo = jnp.where(m0, w0, jnp.where(i0 == _f(-1.0), w1, _f(0.0)))
            whi = jnp.where((i0 >= _f(0.0)) & (i0 <= _f(126.0)), w1, _f(0.0))
            r = jnp.minimum(jnp.maximum(i0, _f(0.0)),
                            _f(127.0)).astype(jnp.int32)
            return wlo, whi, r

        def stage1(slot, base):
            zb = base >> 14
            sz = (_f(2.0) * zb.astype(jnp.float32) + _f(1.0)) \
                * _f(1.0 / 128.0) - _f(1.0)
            sz_v = jnp.full((_L,), sz, jnp.float32)

            @plsc.parallel_loop(0, _NVR, 1, unroll=_UNROLL)
            def _(j):
                off = j * _L
                v0 = base + off
                xb = v0 & 127
                yb = (v0 >> 7) & 127
                sx = (_f(2.0) * xb.astype(jnp.float32) + _f(1.0)) \
                    * _f(1.0 / 128.0) - _f(1.0)
                sy = (_f(2.0) * yb.astype(jnp.float32) + _f(1.0)) \
                    * _f(1.0 / 128.0) - _f(1.0)
                sx_v = jnp.full((_L,), sx, jnp.float32) + lane_s
                sy_v = jnp.full((_L,), sy, jnp.float32)

                ix0, wx0, wx1 = coord(sx_v, flow_v[slot, 0, pl.ds(off, _L)])
                iy0, wy0, wy1 = coord(sy_v, flow_v[slot, 1, pl.ds(off, _L)])
                iz0, wz0, wz1 = coord(sz_v, flow_v[slot, 2, pl.ds(off, _L)])

                wxlo, wxhi, xr = lohi(ix0, wx0, wx1)
                wylo, wyhi, yr = lohi(iy0, wy0, wy1)

                wz0m = jnp.where((iz0 >= _f(0.0)) & (iz0 <= _f(127.0)),
                                 wz0, _f(0.0))
                wz1m = jnp.where((iz0 >= _f(-1.0)) & (iz0 <= _f(126.0)),
                                 wz1, _f(0.0))
                zr0 = jnp.minimum(jnp.maximum(iz0, _f(0.0)),
                                  _f(127.0)).astype(jnp.int32)
                zr1 = jnp.minimum(jnp.maximum(iz0 + _f(1.0), _f(0.0)),
                                  _f(127.0)).astype(jnp.int32)

                rbase = (yr << 7) + xr
                idx_v[slot, pl.ds(off, _L)] = (zr0 << 14) + rbase
                idx_v[slot, pl.ds(_CH + off, _L)] = (zr1 << 14) + rbase

                pq = (wylo * wxlo, wylo * wxhi, wyhi * wxlo, wyhi * wxhi)
                for q in range(4):
                    w_v[slot, q, pl.ds(off, _L)] = wz0m * pq[q]
                    w_v[slot, 4 + q, pl.ds(off, _L)] = wz1m * pq[q]

        def stage2(slot):
            @plsc.parallel_loop(0, _NVR, 1, unroll=_UNROLL2)
            def _(j):
                off = j * _L
                acc = [jnp.zeros((_L,), jnp.float32) for _ in range(_C)]
                for s in range(2):
                    ridx = s * _CH + off + lanes
                    for q in range(4):
                        w = w_v[slot, s * 4 + q, pl.ds(off, _L)]
                        for h in range(2):
                            wrd = plsc.load_gather(
                                rows_v.at[slot],
                                [ridx, jnp.full((_L,), q * 2 + h, jnp.int32)])
                            va, vb = plsc.unpack(
                                plsc.bitcast(wrd, jnp.bfloat16),
                                format=plsc.PackFormat.INTERLEAVED)
                            acc[2 * h] = acc[2 * h] + w * va
                            acc[2 * h + 1] = acc[2 * h + 1] + w * vb
                for c in range(_C):
                    out_v[slot & 1, c, pl.ds(off, _L)] = acc[c]

        def fire_flow(t, slot):
            base = vbase + t * _CH
            pltpu.async_copy(flow_hbm.at[:, pl.ds(base, _CH)],
                             flow_v.at[slot], sem_f.at[slot])

        def fire_gather(slot):
            pltpu.async_copy(table_hbm.at[idx_v.at[slot]], rows_v.at[slot],
                             sem_g.at[slot])

        def wait_flow(slot):
            pltpu.make_async_copy(flow_hbm.at[:, pl.ds(0, _CH)],
                                  flow_v.at[slot], sem_f.at[slot]).wait()

        def wait_gather(slot):
            pltpu.make_async_copy(table_hbm.at[idx_v.at[slot]],
                                  rows_v.at[slot], sem_g.at[slot]).wait()

        def finish(t, slot):
            wait_gather(slot)
            stage2(slot)
            base = vbase + t * _CH
            pltpu.sync_copy(out_v.at[slot & 1],
                            out_hbm.at[:, pl.ds(base, _CH)])

        for i in range(_DEPTH):
            fire_flow(i, i)
        for i in range(_DEPTH - 1):
            wait_flow(i)
            stage1(i, vbase + i * _CH)
            fire_gather(i)

        def chunk_body(t, _):
            s = t % _DEPTH
            ps = (t + 1) % _DEPTH
            wait_flow(s)
            stage1(s, vbase + t * _CH)
            fire_gather(s)
            finish(t - (_DEPTH - 1), ps)
            @pl.when(t + 1 < _NCHUNK)
            def _():
                fire_flow(t + 1, (t + 1) % _DEPTH)
            return ()

        lax.fori_loop(_DEPTH - 1, _NCHUNK, chunk_body, (), unroll=False)
        for r in range(_NCHUNK - _DEPTH + 1, _NCHUNK):
            finish(r, r % _DEPTH)

    return k(table, flow3)


def kernel(src, flow):
    src2d = src.reshape(_C, _N)
    flow3 = flow.reshape(3, _N)
    table = _sc_build_table(src2d).reshape(_N, 8)
    out = _sc_warp(table, flow3)
    return out.reshape(1, _C, _D, _H, _W)


# speedup vs baseline: 1.6325x; 1.0381x over previous
import functools

import jax
import jax.numpy as jnp
from jax import lax
from jax.experimental import pallas as pl
from jax.experimental.pallas import tpu as pltpu
from jax.experimental.pallas import tpu_sc as plsc

_D = _H = _W = 128
_C = 4
_N = _D * _H * _W
_NC, _NS, _L = 2, 16, 16
_NW = _NC * _NS
_VPW = _N // _NW
_CH = 1024
_NCHUNK = _VPW // _CH
_NVR = _CH // _L
_NIDX = 2 * _CH
_DEPTH = 3
_BCH = 1024
_BNCHUNK = _VPW // _BCH
_BNVR = _BCH // _L
_UNROLL = 4
_UNROLL2 = 2
_BCHP = _BCH + 144


def _f(x):
    return jnp.float32(x)


def _sc_build_table(src2d):
    mesh = plsc.VectorSubcoreMesh(core_axis_name="c", subcore_axis_name="s",
                                  num_cores=_NC, num_subcores=_NS)

    @functools.partial(
        pl.kernel,
        out_type=jax.ShapeDtypeStruct((_N * 8,), jnp.float32),
        mesh=mesh,
        compiler_params=pltpu.CompilerParams(needs_layout_passes=False,
                                             use_tc_tiling_on_sc=False),
        scratch_types=[
            pltpu.VMEM((2, _C, _BCHP), jnp.float32),
            pltpu.VMEM((2, _BCH * 8), jnp.float32),
            pltpu.SemaphoreType.DMA((2,)),
            pltpu.SemaphoreType.DMA((2,)),
        ],
    )
    def k(src_hbm, tab_hbm, sbuf, tbuf, sem_i, sem_o):
        wid = lax.axis_index("s") * _NC + lax.axis_index("c")
        vbase = wid * _VPW
        lanes = lax.iota(jnp.int32, _L)
        lanes8 = lanes * 8

        def fire_in(t, slot):
            base = vbase + t * _BCH
            sbase = jnp.minimum(base, _N - _BCHP)
            pltpu.async_copy(src_hbm.at[:, pl.ds(sbase, _BCHP)],
                             sbuf.at[slot], sem_i.at[slot])

        def wait_in(slot):
            pltpu.make_async_copy(src_hbm.at[:, pl.ds(0, _BCHP)],
                                  sbuf.at[slot], sem_i.at[slot]).wait()

        def fire_out(t, slot):
            base = vbase + t * _BCH
            pltpu.async_copy(tbuf.at[slot],
                             tab_hbm.at[pl.ds(base * 8, _BCH * 8)],
                             sem_o.at[slot])

        def wait_out(slot):
            pltpu.make_async_copy(tbuf.at[slot],
                                  tab_hbm.at[pl.ds(0, _BCH * 8)],
                                  sem_o.at[slot]).wait()

        def scatter(t, slot):
            base = vbase + t * _BCH
            is_tail = base > _N - _BCHP
            shift = base - jnp.minimum(base, _N - _BCHP)

            @pl.when(jnp.logical_not(is_tail))
            def _():
                @plsc.parallel_loop(0, _BNVR, 1, unroll=_UNROLL)
                def _(j):
                    off = j * _L
                    ib = off * 8 + lanes8
                    for q, dlt in enumerate((0, 1, 128, 129)):
                        vs = [sbuf[slot, c, pl.ds(off + dlt, _L)]
                              for c in range(_C)]
                        for h in range(2):
                            pk = plsc.bitcast(
                                plsc.pack(vs[2 * h], vs[2 * h + 1],
                                          format=plsc.PackFormat.INTERLEAVED),
                                jnp.float32)
                            plsc.store_scatter(tbuf.at[slot],
                                               [ib + (q * 2 + h)], pk)

            @pl.when(is_tail)
            def _():
                @plsc.parallel_loop(0, _BNVR, 1, unroll=1)
                def _(j):
                    off = j * _L
                    ib = off * 8 + lanes8
                    for q, dlt in enumerate((0, 1, 128, 129)):
                        hidx = jnp.minimum(shift + off + dlt + lanes,
                                           _BCHP - 1)
                        vs = [plsc.load_gather(sbuf.at[slot, c], [hidx])
                              for c in range(_C)]
                        for h in range(2):
                            pk = plsc.bitcast(
                                plsc.pack(vs[2 * h], vs[2 * h + 1],
                                          format=plsc.PackFormat.INTERLEAVED),
                                jnp.float32)
                            plsc.store_scatter(tbuf.at[slot],
                                               [ib + (q * 2 + h)], pk)

        fire_in(0, 0)
        fire_in(1, 1)
        wait_in(0)
        scatter(0, 0)
        fire_out(0, 0)

        def body(t, _):
            s = t & 1
            ns = 1 - s
            wait_in(s)

            @pl.when(t >= 2)
            def _():
                wait_out(s)
            scatter(t, s)
            fire_out(t, s)

            @pl.when(t + 1 < _BNCHUNK)
            def _():
                fire_in(t + 1, ns)
            return ()

        lax.fori_loop(1, _BNCHUNK, body, (), unroll=False)
        wait_out((_BNCHUNK - 2) & 1)
        wait_out((_BNCHUNK - 1) & 1)

    return k(src2d)


def _sc_warp(table, flow3):
    mesh = plsc.VectorSubcoreMesh(core_axis_name="c", subcore_axis_name="s",
                                  num_cores=_NC, num_subcores=_NS)

    @functools.partial(
        pl.kernel,
        out_type=jax.ShapeDtypeStruct((_C, _N), jnp.float32),
        mesh=mesh,
        compiler_params=pltpu.CompilerParams(needs_layout_passes=False,
                                             use_tc_tiling_on_sc=False),
        scratch_types=[
            pltpu.VMEM((_DEPTH, 3, _CH), jnp.float32),
            pltpu.VMEM((_DEPTH, _NIDX), jnp.int32),
            pltpu.VMEM((_DEPTH, _NIDX, 8), jnp.float32),
            pltpu.VMEM((_DEPTH, 8, _CH), jnp.float32),
            pltpu.VMEM((2, _C, _CH), jnp.float32),
            pltpu.SemaphoreType.DMA((_DEPTH,)),
            pltpu.SemaphoreType.DMA((_DEPTH,)),
        ],
    )
    def k(table_hbm, flow_hbm, out_hbm, flow_v, idx_v, rows_v, w_v, out_v,
          sem_f, sem_g):
        wid = lax.axis_index("s") * _NC + lax.axis_index("c")
        vbase = wid * _VPW
        lanes = lax.iota(jnp.int32, _L)
        lanes_f = lanes.astype(jnp.float32)
        lane_s = lanes_f * _f(1.0 / 64.0)

        def coord(sbase, fl):
            x = sbase + fl
            i = ((x + _f(1.0)) * _f(128.0) - _f(1.0)) * _f(0.5)
            ic = jnp.minimum(jnp.maximum(i, _f(-4.0)), _f(132.0))
            t = ic.astype(jnp.int32).astype(jnp.float32)
            i0 = t - jnp.where(t > ic, _f(1.0), _f(0.0))
            w1 = ic - i0
            w0 = _f(1.0) - w1
            return i0, w0, w1

        def lohi(i0, w0, w1):
            m0 = (i0 >= _f(0.0)) & (i0 <= _f(127.0))
            wlo = jnp.where(m0, w0, jnp.where(i0 == _f(-1.0), w1, _f(0.0)))
            whi = jnp.where((i0 >= _f(0.0)) & (i0 <= _f(126.0)), w1, _f(0.0))
            r = jnp.minimum(jnp.maximum(i0, _f(0.0)),
                            _f(127.0)).astype(jnp.int32)
            return wlo, whi, r

        def stage1(slot, base):
            zb = base >> 14
            sz = (_f(2.0) * zb.astype(jnp.float32) + _f(1.0)) \
                * _f(1.0 / 128.0) - _f(1.0)
            sz_v = jnp.full((_L,), sz, jnp.float32)

            @plsc.parallel_loop(0, _NVR, 1, unroll=_UNROLL)
            def _(j):
                off = j * _L
                v0 = base + off
                xb = v0 & 127
                yb = (v0 >> 7) & 127
                sx = (_f(2.0) * xb.astype(jnp.float32) + _f(1.0)) \
                    * _f(1.0 / 128.0) - _f(1.0)
                sy = (_f(2.0) * yb.astype(jnp.float32) + _f(1.0)) \
                    * _f(1.0 / 128.0) - _f(1.0)
                sx_v = jnp.full((_L,), sx, jnp.float32) + lane_s
                sy_v = jnp.full((_L,), sy, jnp.float32)

                ix0, wx0, wx1 = coord(sx_v, flow_v[slot, 0, pl.ds(off, _L)])
                iy0, wy0, wy1 = coord(sy_v, flow_v[slot, 1, pl.ds(off, _L)])
                iz0, wz0, wz1 = coord(sz_v, flow_v[slot, 2, pl.ds(off, _L)])

                wxlo, wxhi, xr = lohi(ix0, wx0, wx1)
                wylo, wyhi, yr = lohi(iy0, wy0, wy1)

                wz0m = jnp.where((iz0 >= _f(0.0)) & (iz0 <= _f(127.0)),
                                 wz0, _f(0.0))
                wz1m = jnp.where((iz0 >= _f(-1.0)) & (iz0 <= _f(126.0)),
                                 wz1, _f(0.0))
                zr0 = jnp.minimum(jnp.maximum(iz0, _f(0.0)),
                                  _f(127.0)).astype(jnp.int32)
                zr1 = jnp.minimum(jnp.maximum(iz0 + _f(1.0), _f(0.0)),
                                  _f(127.0)).astype(jnp.int32)

                rbase = (yr << 7) + xr
                idx_v[slot, pl.ds(off, _L)] = (zr0 << 14) + rbase
                idx_v[slot, pl.ds(_CH + off, _L)] = (zr1 << 14) + rbase

                pq = (wylo * wxlo, wylo * wxhi, wyhi * wxlo, wyhi * wxhi)
                for q in range(4):
                    w_v[slot, q, pl.ds(off, _L)] = wz0m * pq[q]
                    w_v[slot, 4 + q, pl.ds(off, _L)] = wz1m * pq[q]

        def stage2(slot):
            @plsc.parallel_loop(0, _NVR, 1, unroll=_UNROLL2)
            def _(j):
                off = j * _L
                acc = [jnp.zeros((_L,), jnp.float32) for _ in range(_C)]
                for s in range(2):
                    ridx = s * _CH + off + lanes
                    for q in range(4):
                        w = w_v[slot, s * 4 + q, pl.ds(off, _L)]
                        for h in range(2):
                            wrd = plsc.load_gather(
                                rows_v.at[slot],
                                [ridx, jnp.full((_L,), q * 2 + h, jnp.int32)])
                            va, vb = plsc.unpack(
                                plsc.bitcast(wrd, jnp.bfloat16),
                                format=plsc.PackFormat.INTERLEAVED)
                            acc[2 * h] = acc[2 * h] + w * va
                            acc[2 * h + 1] = acc[2 * h + 1] + w * vb
                for c in range(_C):
                    out_v[slot & 1, c, pl.ds(off, _L)] = acc[c]

        def fire_flow(t, slot):
            base = vbase + t * _CH
            pltpu.async_copy(flow_hbm.at[:, pl.ds(base, _CH)],
                             flow_v.at[slot], sem_f.at[slot])

        def fire_gather(slot):
            pltpu.async_copy(table_hbm.at[idx_v.at[slot]], rows_v.at[slot],
                             sem_g.at[slot])

        def wait_flow(slot):
            pltpu.make_async_copy(flow_hbm.at[:, pl.ds(0, _CH)],
                                  flow_v.at[slot], sem_f.at[slot]).wait()

        def wait_gather(slot):
            pltpu.make_async_copy(table_hbm.at[idx_v.at[slot]],
                                  rows_v.at[slot], sem_g.at[slot]).wait()

        def finish(t, slot):
            wait_gather(slot)
            stage2(slot)
            base = vbase + t * _CH
            pltpu.sync_copy(out_v.at[slot & 1],
                            out_hbm.at[:, pl.ds(base, _CH)])

        for i in range(_DEPTH):
            fire_flow(i, i)
        for i in range(_DEPTH - 1):
            wait_flow(i)
            stage1(i, vbase + i * _CH)
            fire_gather(i)

        def chunk_body(t, _):
            s = t % _DEPTH
            ps = (t + 1) % _DEPTH
            wait_flow(s)
            stage1(s, vbase + t * _CH)
            fire_gather(s)
            finish(t - (_DEPTH - 1), ps)
            @pl.when(t + 1 < _NCHUNK)
            def _():
                fire_flow(t + 1, (t + 1) % _DEPTH)
            return ()

        lax.fori_loop(_DEPTH - 1, _NCHUNK, chunk_body, (), unroll=False)
        for r in range(_NCHUNK - _DEPTH + 1, _NCHUNK):
            finish(r, r % _DEPTH)

    return k(table, flow3)


def kernel(src, flow):
    src2d = src.reshape(_C, _N)
    flow3 = flow.reshape(3, _N)
    table = _sc_build_table(src2d).reshape(_N, 8)
    out = _sc_warp(table, flow3)
    return out.reshape(1, _C, _D, _H, _W)
